# Initial kernel scaffold; baseline (speedup 1.0000x reference)
import jax
import jax.numpy as jnp
from jax.experimental import pallas as pl


def kernel(xcat, xfeat, table, W_phi, b_phi, W_rho, b_rho, W_out, b_out):
    raise NotImplementedError("write your pallas kernel here")


# speedup vs baseline: 16.3347x; 16.3347x over previous
import functools

import jax
import jax.numpy as jnp
from jax import lax
from jax.experimental import pallas as pl
from jax.experimental.pallas import tpu as pltpu
from jax.experimental.pallas import tpu_sc as plsc


@functools.partial(jax.jit, static_argnums=(2, 3))
def _sc_gather(table, idx, n_rows, emb_dim):
    info = plsc.get_sparse_core_info()
    nc, ns = info.num_cores, info.num_subcores
    nw = nc * ns
    rows_per_w = n_rows // nw
    SUB = 128
    CHUNK = 512
    n_sub = CHUNK // SUB
    n_chunks = rows_per_w // CHUNK
    n_pairs = n_chunks // 2
    assert rows_per_w % CHUNK == 0 and CHUNK % SUB == 0 and n_chunks % 2 == 0

    mesh = plsc.VectorSubcoreMesh(core_axis_name="c", subcore_axis_name="s")

    @functools.partial(
        pl.kernel,
        mesh=mesh,
        compiler_params=pltpu.CompilerParams(use_tc_tiling_on_sc=False),
        out_type=jax.ShapeDtypeStruct((n_rows, emb_dim), jnp.float32),
        scratch_types=[
            pltpu.VMEM((rows_per_w,), jnp.int32),
            pltpu.VMEM((2, CHUNK, emb_dim), jnp.float32),
            pltpu.SemaphoreType.DMA,
            pltpu.SemaphoreType.DMA,
            pltpu.SemaphoreType.DMA,
            pltpu.SemaphoreType.DMA,
        ],
    )
    def gather_kernel(table_hbm, idx_hbm, out_hbm, idx_v, rows_v, g0, g1, w0, w1):
        wid = lax.axis_index("s") * nc + lax.axis_index("c")
        base = wid * rows_per_w
        pltpu.sync_copy(idx_hbm.at[pl.ds(base, rows_per_w)], idx_v)

        gsems = (g0, g1)
        wsems = (w0, w1)

        def fire(o, buf):
            for j in range(n_sub):
                pltpu.async_copy(
                    table_hbm.at[idx_v.at[pl.ds(o * CHUNK + j * SUB, SUB)]],
                    rows_v.at[buf, pl.ds(j * SUB, SUB)],
                    gsems[buf],
                )

        def drain_gathers(buf):
            for j in range(n_sub):
                pltpu.make_async_copy(
                    table_hbm.at[idx_v.at[pl.ds(j * SUB, SUB)]],
                    rows_v.at[buf, pl.ds(j * SUB, SUB)],
                    gsems[buf],
                ).wait()

        def writeout(o, buf):
            pltpu.async_copy(
                rows_v.at[buf],
                out_hbm.at[pl.ds(base + o * CHUNK, CHUNK)],
                wsems[buf],
            )

        def drain_writeout(buf):
            pltpu.make_async_copy(
                rows_v.at[buf],
                out_hbm.at[pl.ds(base, CHUNK)],
                wsems[buf],
            ).wait()

        fire(0, 0)

        def body(p, _):
            o0 = 2 * p
            o1 = o0 + 1
            drain_gathers(0)

            @pl.when(p > 0)
            def _():
                drain_writeout(1)
            fire(o1, 1)
            writeout(o0, 0)
            drain_gathers(1)
            drain_writeout(0)

            @pl.when(p + 1 < n_pairs)
            def _():
                fire(o1 + 1, 0)
            writeout(o1, 1)
            return 0

        lax.fori_loop(0, n_pairs, body, 0)
        drain_writeout(1)

    return gather_kernel(table, idx)


def _tc_mlp(g, fcol, W1, wf, b_phi, W_rho, b_rho, W_out, b_out, batch, setlen,
            bb):
    n_rows, emb_dim = g.shape
    hid = W1.shape[1]
    grid = batch // bb
    bbl = bb * setlen

    def body(g_ref, f_ref, w1_ref, wf_ref, bphi_ref, wrho_ref, brho_ref,
             wout_ref, bout_ref, o_ref):
        e = jnp.maximum(g_ref[...], 0.0)
        z = lax.dot_general(e, w1_ref[...], (((1,), (0,)), ((), ())),
                            preferred_element_type=jnp.float32)
        z = z + f_ref[...] * wf_ref[...] + bphi_ref[...]
        h = jnp.maximum(z, 0.0)
        pooled = jnp.sum(h.reshape(bb, setlen, hid), axis=1)
        s = lax.dot_general(pooled, wrho_ref[...], (((1,), (0,)), ((), ())),
                            preferred_element_type=jnp.float32)
        s = jnp.maximum(s + brho_ref[...], 0.0)
        o = lax.dot_general(s, wout_ref[...], (((1,), (0,)), ((), ())),
                            preferred_element_type=jnp.float32)
        o_ref[...] = o + bout_ref[...]

    full = lambda shape: pl.BlockSpec(shape, lambda i: (0, 0))
    return pl.pallas_call(
        body,
        grid=(grid,),
        in_specs=[
            pl.BlockSpec((bbl, emb_dim), lambda i: (i, 0)),
            pl.BlockSpec((bbl, 1), lambda i: (i, 0)),
            full(W1.shape),
            full(wf.shape),
            full(b_phi.shape),
            full(W_rho.shape),
            full(b_rho.shape),
            full(W_out.shape),
            full(b_out.shape),
        ],
        out_specs=pl.BlockSpec((bb, 1), lambda i: (i, 0)),
        out_shape=jax.ShapeDtypeStruct((batch, 1), jnp.float32),
    )(g, fcol, W1, wf, b_phi, W_rho, b_rho, W_out, b_out)


def kernel(xcat, xfeat, table, W_phi, b_phi, W_rho, b_rho, W_out, b_out):
    batch, setlen = xcat.shape
    nembed, emb_dim = table.shape
    hid = W_phi.shape[1]
    n_rows = batch * setlen

    idx = xcat.reshape(n_rows).astype(jnp.int32)
    g = _sc_gather(table, idx, n_rows, emb_dim)

    fcol = xfeat.reshape(n_rows, 1)
    W1 = W_phi[:emb_dim, :]
    wf = W_phi[emb_dim:emb_dim + 1, :]
    out = _tc_mlp(g, fcol, W1, wf, b_phi.reshape(1, hid), W_rho,
                  b_rho.reshape(1, hid), W_out, b_out.reshape(1, 1),
                  batch, setlen, bb=128)
    return out


# speedup vs baseline: 23.9414x; 1.4657x over previous
import functools

import jax
import jax.numpy as jnp
from jax import lax
from jax.experimental import pallas as pl
from jax.experimental.pallas import tpu as pltpu
from jax.experimental.pallas import tpu_sc as plsc


@functools.partial(jax.jit, static_argnums=(2, 3))
def _sc_gather(table, idx, n_rows, emb_dim):
    info = plsc.get_sparse_core_info()
    nc, ns = info.num_cores, info.num_subcores
    nw = nc * ns
    rows_per_w = n_rows // nw
    SUB = 128
    CHUNK = 512
    n_sub = CHUNK // SUB
    n_chunks = rows_per_w // CHUNK
    n_pairs = n_chunks // 2
    assert rows_per_w % CHUNK == 0 and CHUNK % SUB == 0 and n_chunks % 2 == 0

    mesh = plsc.VectorSubcoreMesh(core_axis_name="c", subcore_axis_name="s")

    @functools.partial(
        pl.kernel,
        mesh=mesh,
        compiler_params=pltpu.CompilerParams(use_tc_tiling_on_sc=False),
        out_type=jax.ShapeDtypeStruct((n_rows, emb_dim), jnp.float32),
        scratch_types=[
            pltpu.VMEM((rows_per_w,), jnp.int32),
            pltpu.VMEM((2, CHUNK, emb_dim), jnp.float32),
            pltpu.SemaphoreType.DMA,
            pltpu.SemaphoreType.DMA,
            pltpu.SemaphoreType.DMA,
            pltpu.SemaphoreType.DMA,
        ],
    )
    def gather_kernel(table_hbm, idx_hbm, out_hbm, idx_v, rows_v, g0, g1, w0, w1):
        wid = lax.axis_index("s") * nc + lax.axis_index("c")
        base = wid * rows_per_w
        pltpu.sync_copy(idx_hbm.at[pl.ds(base, rows_per_w)], idx_v)

        gsems = (g0, g1)
        wsems = (w0, w1)

        def fire(o, buf):
            for j in range(n_sub):
                pltpu.async_copy(
                    table_hbm.at[idx_v.at[pl.ds(o * CHUNK + j * SUB, SUB)]],
                    rows_v.at[buf, pl.ds(j * SUB, SUB)],
                    gsems[buf],
                )

        def drain_gathers(buf):
            for j in range(n_sub):
                pltpu.make_async_copy(
                    table_hbm.at[idx_v.at[pl.ds(j * SUB, SUB)]],
                    rows_v.at[buf, pl.ds(j * SUB, SUB)],
                    gsems[buf],
                ).wait()

        def writeout(o, buf):
            pltpu.async_copy(
                rows_v.at[buf],
                out_hbm.at[pl.ds(base + o * CHUNK, CHUNK)],
                wsems[buf],
            )

        def drain_writeout(buf):
            pltpu.make_async_copy(
                rows_v.at[buf],
                out_hbm.at[pl.ds(base, CHUNK)],
                wsems[buf],
            ).wait()

        fire(0, 0)

        def body(p, _):
            o0 = 2 * p
            o1 = o0 + 1
            drain_gathers(0)

            @pl.when(p > 0)
            def _():
                drain_writeout(1)
            fire(o1, 1)
            writeout(o0, 0)
            drain_gathers(1)
            drain_writeout(0)

            @pl.when(p + 1 < n_pairs)
            def _():
                fire(o1 + 1, 0)
            writeout(o1, 1)
            return 0

        lax.fori_loop(0, n_pairs, body, 0)
        drain_writeout(1)

    return gather_kernel(table, idx)


def _tc_mlp(g128, f4T, W_bd, wf, b_phi, W_rho, b_rho, W_out, b_out, batch,
            setlen, bb, pack):
    hid = W_rho.shape[0]
    grid = batch // bb
    bbl = bb * setlen
    gb = bbl // pack
    per = setlen // pack

    def body(g_ref, f_ref, wbd_ref, wf_ref, bphi_ref, wrho_ref, brho_ref,
             wout_ref, bout_ref, o_ref):
        e4 = jnp.maximum(g_ref[...], 0.0)
        z4 = lax.dot_general(e4, wbd_ref[...], (((1,), (0,)), ((), ())),
                             preferred_element_type=jnp.float32)
        f4 = jnp.transpose(f_ref[...])
        wf_row = wf_ref[...]
        bphi = bphi_ref[...]
        hsum = None
        for j in range(pack):
            zj = (z4[:, j * hid:(j + 1) * hid]
                  + f4[:, j:j + 1] * wf_row + bphi)
            hj = jnp.maximum(zj, 0.0)
            hsum = hj if hsum is None else hsum + hj
        pooled = jnp.sum(hsum.reshape(bb, per, hid), axis=1)
        s = lax.dot_general(pooled, wrho_ref[...], (((1,), (0,)), ((), ())),
                            preferred_element_type=jnp.float32)
        s = jnp.maximum(s + brho_ref[...], 0.0)
        o = lax.dot_general(s, wout_ref[...], (((1,), (0,)), ((), ())),
                            preferred_element_type=jnp.float32)
        o_ref[...] = o + bout_ref[...]

    full = lambda shape: pl.BlockSpec(shape, lambda i: (0, 0))
    return pl.pallas_call(
        body,
        grid=(grid,),
        in_specs=[
            pl.BlockSpec((gb, 128), lambda i: (i, 0)),
            pl.BlockSpec((pack, gb), lambda i: (0, i)),
            full(W_bd.shape),
            full(wf.shape),
            full(b_phi.shape),
            full(W_rho.shape),
            full(b_rho.shape),
            full(W_out.shape),
            full(b_out.shape),
        ],
        out_specs=pl.BlockSpec((bb, 1), lambda i: (i, 0)),
        out_shape=jax.ShapeDtypeStruct((batch, 1), jnp.float32),
    )(g128, f4T, W_bd, wf, b_phi, W_rho, b_rho, W_out, b_out)


def kernel(xcat, xfeat, table, W_phi, b_phi, W_rho, b_rho, W_out, b_out):
    batch, setlen = xcat.shape
    nembed, emb_dim = table.shape
    hid = W_phi.shape[1]
    n_rows = batch * setlen
    pack = 128 // emb_dim

    idx = xcat.reshape(n_rows).astype(jnp.int32)
    g = _sc_gather(table, idx, n_rows, emb_dim)
    g128 = g.reshape(n_rows // pack, 128)

    f4T = xfeat.reshape(n_rows // pack, pack).T
    W1 = W_phi[:emb_dim, :]
    wf = W_phi[emb_dim:emb_dim + 1, :]
    W_bd = jnp.zeros((128, pack * hid), jnp.float32)
    for j in range(pack):
        W_bd = W_bd.at[j * emb_dim:(j + 1) * emb_dim, j * hid:(j + 1) * hid].set(W1)

    out = _tc_mlp(g128, f4T, W_bd, wf, b_phi.reshape(1, hid), W_rho,
                  b_rho.reshape(1, hid), W_out, b_out.reshape(1, 1),
                  batch, setlen, bb=128, pack=pack)
    return out


# speedup vs baseline: 25.0582x; 1.0466x over previous
import functools

import jax
import jax.numpy as jnp
from jax import lax
from jax.experimental import pallas as pl
from jax.experimental.pallas import tpu as pltpu
from jax.experimental.pallas import tpu_sc as plsc


@functools.partial(jax.jit, static_argnums=(2, 3))
def _sc_gather(table, idx, n_rows, emb_dim):
    info = plsc.get_sparse_core_info()
    nc, ns = info.num_cores, info.num_subcores
    nw = nc * ns
    rows_per_w = n_rows // nw
    SUB = 128
    CHUNK = 128
    n_sub = max(1, CHUNK // SUB)
    SUB = min(SUB, CHUNK)
    n_chunks = rows_per_w // CHUNK
    n_pairs = n_chunks // 2
    assert rows_per_w % CHUNK == 0 and CHUNK % SUB == 0 and n_chunks % 2 == 0

    mesh = plsc.VectorSubcoreMesh(core_axis_name="c", subcore_axis_name="s")

    @functools.partial(
        pl.kernel,
        mesh=mesh,
        compiler_params=pltpu.CompilerParams(use_tc_tiling_on_sc=False),
        out_type=jax.ShapeDtypeStruct((n_rows, emb_dim), jnp.float32),
        scratch_types=[
            pltpu.VMEM((rows_per_w,), jnp.int32),
            pltpu.VMEM((2, CHUNK, emb_dim), jnp.float32),
            pltpu.SemaphoreType.DMA,
            pltpu.SemaphoreType.DMA,
            pltpu.SemaphoreType.DMA,
            pltpu.SemaphoreType.DMA,
        ],
    )
    def gather_kernel(table_hbm, idx_hbm, out_hbm, idx_v, rows_v, g0, g1, w0, w1):
        wid = lax.axis_index("s") * nc + lax.axis_index("c")
        base = wid * rows_per_w
        pltpu.sync_copy(idx_hbm.at[pl.ds(base, rows_per_w)], idx_v)

        gsems = (g0, g1)
        wsems = (w0, w1)

        def fire(o, buf):
            for j in range(n_sub):
                pltpu.async_copy(
                    table_hbm.at[idx_v.at[pl.ds(o * CHUNK + j * SUB, SUB)]],
                    rows_v.at[buf, pl.ds(j * SUB, SUB)],
                    gsems[buf],
                )

        def drain_gathers(buf):
            for j in range(n_sub):
                pltpu.make_async_copy(
                    table_hbm.at[idx_v.at[pl.ds(j * SUB, SUB)]],
                    rows_v.at[buf, pl.ds(j * SUB, SUB)],
                    gsems[buf],
                ).wait()

        def writeout(o, buf):
            pltpu.async_copy(
                rows_v.at[buf],
                out_hbm.at[pl.ds(base + o * CHUNK, CHUNK)],
                wsems[buf],
            )

        def drain_writeout(buf):
            pltpu.make_async_copy(
                rows_v.at[buf],
                out_hbm.at[pl.ds(base, CHUNK)],
                wsems[buf],
            ).wait()

        fire(0, 0)

        def body(p, _):
            o0 = 2 * p
            o1 = o0 + 1
            drain_gathers(0)

            @pl.when(p > 0)
            def _():
                drain_writeout(1)
            fire(o1, 1)
            writeout(o0, 0)
            drain_gathers(1)
            drain_writeout(0)

            @pl.when(p + 1 < n_pairs)
            def _():
                fire(o1 + 1, 0)
            writeout(o1, 1)
            return 0

        lax.fori_loop(0, n_pairs, body, 0)
        drain_writeout(1)

    return gather_kernel(table, idx)


def _tc_mlp(g128, f4T, W_bd, wf, b_phi, W_rho, b_rho, W_out, b_out, batch,
            setlen, bb, pack):
    hid = W_rho.shape[0]
    grid = batch // bb
    bbl = bb * setlen
    gb = bbl // pack
    per = setlen // pack

    def body(g_ref, f_ref, wbd_ref, wf_ref, bphi_ref, wrho_ref, brho_ref,
             wout_ref, bout_ref, o_ref):
        e4 = jnp.maximum(g_ref[...], 0.0)
        z4 = lax.dot_general(e4, wbd_ref[...], (((1,), (0,)), ((), ())),
                             preferred_element_type=jnp.float32)
        f4 = jnp.transpose(f_ref[...])
        f4 = f4.astype(jnp.bfloat16).astype(jnp.float32)
        wf_row = wf_ref[...].astype(jnp.bfloat16).astype(jnp.float32)
        bphi = bphi_ref[...]
        hsum = None
        for j in range(pack):
            zj = (z4[:, j * hid:(j + 1) * hid]
                  + f4[:, j:j + 1] * wf_row + bphi)
            hj = jnp.maximum(zj, 0.0)
            hsum = hj if hsum is None else hsum + hj
        pooled = jnp.sum(hsum.reshape(bb, per, hid), axis=1)
        s = lax.dot_general(pooled, wrho_ref[...], (((1,), (0,)), ((), ())),
                            preferred_element_type=jnp.float32)
        s = jnp.maximum(s + brho_ref[...], 0.0)
        o = lax.dot_general(s, wout_ref[...], (((1,), (0,)), ((), ())),
                            preferred_element_type=jnp.float32)
        o_ref[...] = o + bout_ref[...]

    full = lambda shape: pl.BlockSpec(shape, lambda i: (0, 0))
    return pl.pallas_call(
        body,
        grid=(grid,),
        in_specs=[
            pl.BlockSpec((gb, 128), lambda i: (i, 0)),
            pl.BlockSpec((pack, gb), lambda i: (0, i)),
            full(W_bd.shape),
            full(wf.shape),
            full(b_phi.shape),
            full(W_rho.shape),
            full(b_rho.shape),
            full(W_out.shape),
            full(b_out.shape),
        ],
        out_specs=pl.BlockSpec((bb, 1), lambda i: (i, 0)),
        out_shape=jax.ShapeDtypeStruct((batch, 1), jnp.float32),
    )(g128, f4T, W_bd, wf, b_phi, W_rho, b_rho, W_out, b_out)


def kernel(xcat, xfeat, table, W_phi, b_phi, W_rho, b_rho, W_out, b_out):
    batch, setlen = xcat.shape
    nembed, emb_dim = table.shape
    hid = W_phi.shape[1]
    n_rows = batch * setlen
    pack = 128 // emb_dim

    idx = xcat.reshape(n_rows).astype(jnp.int32)

    f4T = xfeat.reshape(n_rows // pack, pack).T
    W1 = W_phi[:emb_dim, :]
    wf = W_phi[emb_dim:emb_dim + 1, :]
    W_bd = jnp.zeros((128, pack * hid), jnp.float32)
    for j in range(pack):
        W_bd = W_bd.at[j * emb_dim:(j + 1) * emb_dim, j * hid:(j + 1) * hid].set(W1)

    nseg = 4
    bseg = batch // nseg
    rseg = n_rows // nseg
    outs = []
    for k in range(nseg):
        idx_k = lax.slice_in_dim(idx, k * rseg, (k + 1) * rseg)
        g_k = _sc_gather(table, idx_k, rseg, emb_dim)
        g128_k = g_k.reshape(rseg // pack, 128)
        f4T_k = lax.slice_in_dim(f4T, k * rseg // pack,
                                 (k + 1) * rseg // pack, axis=1)
        outs.append(_tc_mlp(g128_k, f4T_k, W_bd, wf, b_phi.reshape(1, hid),
                            W_rho, b_rho.reshape(1, hid), W_out,
                            b_out.reshape(1, 1), bseg, setlen, bb=min(128, bseg),
                            pack=pack))
    return jnp.concatenate(outs, axis=0)
